# baseline (speedup 1.0000x reference)
import functools

import jax
import jax.numpy as jnp
from jax import lax
from jax.experimental import pallas as pl
from jax.experimental.pallas import tpu as pltpu
from jax.experimental.pallas import tpu_sc as plsc

D_MODEL = 1024
SEQ = 32
ROWS = SEQ + 1
BATCH = 256
LANES = 16
CB = 32

_info = plsc.get_sparse_core_info()
_NC = _info.num_cores
_NS = _info.num_subcores
_NW = _NC * _NS

_mesh = plsc.VectorSubcoreMesh(core_axis_name="c", subcore_axis_name="s")


@functools.partial(
    pl.kernel,
    mesh=_mesh,
    out_type=jax.ShapeDtypeStruct((ROWS, BATCH, D_MODEL), jnp.float32),
    scratch_types=[
        pltpu.VMEM((CB, D_MODEL), jnp.float32),
        pltpu.VMEM((CB, D_MODEL), jnp.float32),
        pltpu.VMEM((CB, D_MODEL), jnp.float32),
        pltpu.VMEM((LANES, D_MODEL), jnp.float32),
        pltpu.VMEM((LANES,), jnp.int32),
        pltpu.VMEM((CB,), jnp.int32),
        pltpu.VMEM((CB,), jnp.int32),
        pltpu.VMEM((CB,), jnp.int32),
        pltpu.SemaphoreType.DMA,
        pltpu.SemaphoreType.DMA,
        pltpu.SemaphoreType.DMA,
        pltpu.SemaphoreType.DMA,
        pltpu.SemaphoreType.DMA,
        pltpu.SemaphoreType.DMA,
        pltpu.SemaphoreType.DMA,
    ],
)
def _sc_embed(x_hbm, pos_hbm, ct_hbm, out_hbm,
              xb0, xb1, xb2, pbuf, pix, ix0, ix1, ix2,
              sx0, sx1, sx2, sp, so0, so1, so2):
    wid = lax.axis_index("s") * _NC + lax.axis_index("c")
    xbufs = (xb0, xb1, xb2)
    xsems = (sx0, sx1, sx2)
    osems = (so0, so1, so2)
    ixbufs = (ix0, ix1, ix2)
    fw = wid >> 3

    iota = lax.iota(jnp.int32, LANES)
    pix[pl.ds(0, LANES)] = jnp.where(
        iota < 8, fw + 4 * iota,
        jnp.where(iota == 8, jnp.int32(SEQ), jnp.int32(0)))
    pos_cp = pltpu.make_async_copy(pos_hbm.at[pix], pbuf, sp)
    pos_cp.start()

    def params(k):
        if k < 8:
            return fw + 4 * k, (wid & 7) * CB
        return jnp.int32(SEQ), wid * CB

    def issue_in(k):
        s, c0 = params(k)
        sl = k % 3

        @pl.when(s == 0)
        def _():
            pltpu.make_async_copy(ct_hbm.at[pl.ds(c0, CB)],
                                  xbufs[sl], xsems[sl]).start()

        @pl.when(s != 0)
        def _():
            ix = ixbufs[sl]
            base = c0 * SEQ + s - 1
            ix[pl.ds(0, LANES)] = iota * SEQ + base
            ix[pl.ds(LANES, LANES)] = iota * SEQ + base + LANES * SEQ
            pltpu.make_async_copy(x_hbm.at[ix], xbufs[sl], xsems[sl]).start()

    def wait_in(k):
        sl = k % 3
        pltpu.make_async_copy(ct_hbm.at[pl.ds(0, CB)],
                              xbufs[sl], xsems[sl]).wait()

    def compute(k):
        xb = xbufs[k % 3]

        def body(c, cc):
            col = c * LANES
            s = pl.ds(col, LANES)
            vp = pbuf[k, s]
            for g in range(0, CB, 8):
                vs = [xb[r, s] for r in range(g, g + 8)]
                vs = [v + vp for v in vs]
                for r, v in zip(range(g, g + 8), vs):
                    xb[r, s] = v
            return cc

        lax.fori_loop(0, D_MODEL // LANES, body, 0)

    def out_copy(k):
        s, c0 = params(k)
        sl = k % 3
        return pltpu.make_async_copy(
            xbufs[sl], out_hbm.at[s, pl.ds(c0, CB)], osems[sl])

    issue_in(0)
    pos_cp.wait()
    for k in range(8):
        if k >= 2:
            out_copy(k - 2).wait()
        if k + 1 < 8:
            issue_in(k + 1)
        elif k + 1 == 8:
            @pl.when(wid < 8)
            def _():
                issue_in(8)
        wait_in(k)
        compute(k)
        out_copy(k).start()
    out_copy(6).wait()
    out_copy(7).wait()

    @pl.when(wid < 8)
    def _():
        wait_in(8)
        compute(8)
        out_copy(8).start()
        out_copy(8).wait()


def kernel(x, pos_table, class_token):
    out = _sc_embed(
        x.reshape(BATCH * SEQ, D_MODEL),
        pos_table,
        class_token.reshape(BATCH, D_MODEL),
    )
    return jnp.transpose(out, (1, 0, 2))


# speedup vs baseline: 1.1172x; 1.1172x over previous
import functools

import jax
import jax.numpy as jnp
from jax import lax
from jax.experimental import pallas as pl
from jax.experimental.pallas import tpu as pltpu
from jax.experimental.pallas import tpu_sc as plsc

D_MODEL = 1024
SEQ = 32
ROWS = SEQ + 1
BATCH = 256
LANES = 16
CB = 32
NTASK = ROWS * (BATCH // CB)

_info = plsc.get_sparse_core_info()
_NC = _info.num_cores
_NS = _info.num_subcores
_NW = _NC * _NS

_mesh = plsc.VectorSubcoreMesh(core_axis_name="c", subcore_axis_name="s")


@functools.partial(
    pl.kernel,
    mesh=_mesh,
    out_type=jax.ShapeDtypeStruct((ROWS, BATCH, D_MODEL), jnp.float32),
    scratch_types=[
        pltpu.VMEM((CB, D_MODEL), jnp.float32),
        pltpu.VMEM((CB, D_MODEL), jnp.float32),
        pltpu.VMEM((CB, D_MODEL), jnp.float32),
        pltpu.VMEM((D_MODEL,), jnp.float32),
        pltpu.VMEM((D_MODEL,), jnp.float32),
        pltpu.VMEM((CB,), jnp.int32),
        pltpu.VMEM((CB,), jnp.int32),
        pltpu.VMEM((CB,), jnp.int32),
        pltpu.SemaphoreType.DMA,
        pltpu.SemaphoreType.DMA,
        pltpu.SemaphoreType.DMA,
        pltpu.SemaphoreType.DMA,
        pltpu.SemaphoreType.DMA,
        pltpu.SemaphoreType.DMA,
        pltpu.SemaphoreType.DMA,
        pltpu.SemaphoreType.DMA,
    ],
)
def _sc_embed(x_hbm, pos_hbm, ct_hbm, out_hbm,
              xb0, xb1, xb2, pb0, pb1, ix0, ix1, ix2,
              sx0, sx1, sx2, sp0, sp1, so0, so1, so2):
    wid = lax.axis_index("s") * _NC + lax.axis_index("c")
    xbufs = (xb0, xb1, xb2)
    xsems = (sx0, sx1, sx2)
    osems = (so0, so1, so2)
    pbufs = (pb0, pb1)
    psems = (sp0, sp1)
    ixbufs = (ix0, ix1, ix2)

    def params(k):
        if k < 8:
            return (wid >> 3) + 4 * k, (wid & 7) * CB
        return jnp.int32(SEQ), wid * CB

    def issue_in(k):
        s, c0 = params(k)
        sl = k % 3
        pltpu.make_async_copy(pos_hbm.at[pl.ds(s * D_MODEL, D_MODEL)],
                              pbufs[k % 2], psems[k % 2]).start()

        @pl.when(s == 0)
        def _():
            pltpu.make_async_copy(ct_hbm.at[pl.ds(c0, CB)],
                                  xbufs[sl], xsems[sl]).start()

        @pl.when(s != 0)
        def _():
            ix = ixbufs[sl]
            base = c0 * SEQ + s - 1
            iota = lax.iota(jnp.int32, LANES)
            ix[pl.ds(0, LANES)] = iota * SEQ + base
            ix[pl.ds(LANES, LANES)] = iota * SEQ + base + LANES * SEQ
            pltpu.make_async_copy(x_hbm.at[ix], xbufs[sl], xsems[sl]).start()

    def wait_in(k):
        sl = k % 3
        pltpu.make_async_copy(ct_hbm.at[pl.ds(0, CB)],
                              xbufs[sl], xsems[sl]).wait()
        pltpu.make_async_copy(pos_hbm.at[pl.ds(0, D_MODEL)],
                              pbufs[k % 2], psems[k % 2]).wait()

    def compute(k):
        xb = xbufs[k % 3]
        pb = pbufs[k % 2]

        def body(c, cc):
            col = c * LANES
            s = pl.ds(col, LANES)
            vp = pb[s]
            for g in range(0, CB, 8):
                vs = [xb[r, s] for r in range(g, g + 8)]
                vs = [v + vp for v in vs]
                for r, v in zip(range(g, g + 8), vs):
                    xb[r, s] = v
            return cc

        lax.fori_loop(0, D_MODEL // LANES, body, 0)

    def out_copy(k):
        s, c0 = params(k)
        sl = k % 3
        return pltpu.make_async_copy(
            xbufs[sl], out_hbm.at[s, pl.ds(c0, CB)], osems[sl])

    issue_in(0)
    for k in range(8):
        if k >= 2:
            out_copy(k - 2).wait()
        if k + 1 < 8:
            issue_in(k + 1)
        elif k + 1 == 8:
            @pl.when(wid < 8)
            def _():
                issue_in(8)
        wait_in(k)
        compute(k)
        out_copy(k).start()
    out_copy(6).wait()
    out_copy(7).wait()

    @pl.when(wid < 8)
    def _():
        wait_in(8)
        compute(8)
        out_copy(8).start()
        out_copy(8).wait()


def kernel(x, pos_table, class_token):
    out = _sc_embed(
        x.reshape(BATCH * SEQ, D_MODEL),
        pos_table.reshape(ROWS * D_MODEL),
        class_token.reshape(BATCH, D_MODEL),
    )
    return jnp.transpose(out, (1, 0, 2))
